# Initial kernel scaffold; baseline (speedup 1.0000x reference)
import jax
import jax.numpy as jnp
from jax.experimental import pallas as pl


def kernel(x, w1, g1, b1, w2, g2, b2, wd, gd, bd):
    raise NotImplementedError("write your pallas kernel here")


# speedup vs baseline: 1.0851x; 1.0851x over previous
import functools

import jax
import jax.numpy as jnp
from jax.experimental import pallas as pl
from jax.experimental.pallas import tpu as pltpu

EPS = 1e-5
_VMEM_LIMIT = 100 * 1024 * 1024


def _conv1_kernel(xe_ref, wc_ref, z_ref, st_ref, *, Nb, Ho, Wo, Ap, Cin):
    R = Nb * Ho * Wo
    z = None
    for dy in range(3):
        for dx in range(3):
            tap = dy * 3 + dx
            plane = (dy % 2) * 2 + (dx % 2)
            r0 = plane * Ap + dy // 2
            c0 = dx // 2
            t = xe_ref[:, r0:r0 + Ho, c0:c0 + Wo, :].reshape(R, Cin)
            d = jnp.dot(t, wc_ref[tap], preferred_element_type=jnp.float32)
            z = d if z is None else z + d
    z_ref[...] = z.astype(z_ref.dtype)
    st_ref[0, 0:1, :] = jnp.sum(z, axis=0, keepdims=True)
    st_ref[0, 1:2, :] = jnp.sum(z * z, axis=0, keepdims=True)
    st_ref[0, 2:8, :] = jnp.zeros_like(st_ref[0, 2:8, :])


def _conv2_kernel(z_ref, w2_ref, bn_ref, s1_ref, z2_ref, st_ref, ypad_ref,
                  *, Nb, Ho, Wo, C, inv_m):
    R = Nb * Ho * Wo
    s1 = jnp.sum(s1_ref[...], axis=0)
    mean = s1[0:1, :] * inv_m
    var = jnp.maximum(s1[1:2, :] * inv_m - mean * mean, 0.0)
    scale = jax.lax.rsqrt(var + EPS) * bn_ref[0:1, :]
    shift = bn_ref[1:2, :] - mean * scale
    y1 = jnp.maximum(z_ref[...].astype(jnp.float32) * scale + shift, 0.0)

    ypad_ref[...] = jnp.zeros_like(ypad_ref)
    ypad_ref[:, 1:1 + Ho, 1:1 + Wo, :] = (
        y1.astype(ypad_ref.dtype).reshape(Nb, Ho, Wo, C))

    z2 = None
    for dy in range(3):
        for dx in range(3):
            t = ypad_ref[:, dy:dy + Ho, dx:dx + Wo, :].reshape(R, C)
            d = jnp.dot(t, w2_ref[dy * 3 + dx],
                        preferred_element_type=jnp.float32)
            z2 = d if z2 is None else z2 + d
    z2_ref[...] = z2.astype(z2_ref.dtype)
    st_ref[0, 0:1, :] = jnp.sum(z2, axis=0, keepdims=True)
    st_ref[0, 1:2, :] = jnp.sum(z2 * z2, axis=0, keepdims=True)
    st_ref[0, 2:8, :] = jnp.zeros_like(st_ref[0, 2:8, :])


def _final_kernel(z2_ref, zd_ref, s1_ref, s2_ref, bn_ref, o_ref, *, inv_m):
    s1 = jnp.sum(s1_ref[...], axis=0)
    s2 = jnp.sum(s2_ref[...], axis=0)

    m2 = s2[0:1, :] * inv_m
    v2 = jnp.maximum(s2[1:2, :] * inv_m - m2 * m2, 0.0)
    sc2 = jax.lax.rsqrt(v2 + EPS) * bn_ref[2:3, :]
    y2 = z2_ref[...].astype(jnp.float32) * sc2 + (bn_ref[3:4, :] - m2 * sc2)

    md = s1[0:1, :] * inv_m
    vd = jnp.maximum(s1[1:2, :] * inv_m - md * md, 0.0)
    scd = jax.lax.rsqrt(vd + EPS) * bn_ref[4:5, :]
    yd = zd_ref[...].astype(jnp.float32) * scd + (bn_ref[5:6, :] - md * scd)

    o_ref[...] = jnp.maximum(y2 + yd, 0.0)


def _block_impl(x, w1, g1, b1, w2, g2, b2, wd, gd, bd):
    N, Cin, H, W = x.shape
    C = w1.shape[-1]
    assert H % 2 == 0 and W % 2 == 0
    assert Cin % 128 == 0 and C % 128 == 0
    Ho, Wo = H // 2, W // 2
    Ap = Ho + 1
    M = N * Ho * Wo
    inv_m = 1.0 / M
    Nb = next(t for t in (8, 4, 2, 1) if N % t == 0)
    nb = N // Nb
    R = Nb * Ho * Wo

    xh = jnp.transpose(x.astype(jnp.bfloat16), (0, 2, 3, 1))
    xp = jnp.pad(xh, ((0, 0), (1, 1), (1, 1), (0, 0)))
    planes = [xp[:, pi::2, pj::2, :][:, :Ap, :Ap, :]
              for pi in (0, 1) for pj in (0, 1)]
    xe = jnp.concatenate(planes, axis=1)

    wc = jnp.zeros((9, Cin, 2 * C), jnp.float32)
    wc = wc.at[:, :, :C].set(w1.reshape(9, Cin, C))
    wc = wc.at[4, :, C:].set(wd.reshape(Cin, C))
    wc = wc.astype(jnp.bfloat16)
    w2p = w2.reshape(9, C, C).astype(jnp.bfloat16)
    zrow = jnp.zeros_like(g1)
    bn = jnp.stack([g1, b1, g2, b2, gd, bd, zrow, zrow])

    par = pltpu.CompilerParams(dimension_semantics=("parallel",),
                               vmem_limit_bytes=_VMEM_LIMIT)

    z, s1b = pl.pallas_call(
        functools.partial(_conv1_kernel, Nb=Nb, Ho=Ho, Wo=Wo, Ap=Ap, Cin=Cin),
        grid=(nb,),
        in_specs=[
            pl.BlockSpec((Nb, 4 * Ap, Ap, Cin), lambda i: (i, 0, 0, 0)),
            pl.BlockSpec((9, Cin, 2 * C), lambda i: (0, 0, 0)),
        ],
        out_specs=[
            pl.BlockSpec((R, 2 * C), lambda i: (i, 0)),
            pl.BlockSpec((1, 8, 2 * C), lambda i: (i, 0, 0)),
        ],
        out_shape=[
            jax.ShapeDtypeStruct((M, 2 * C), jnp.bfloat16),
            jax.ShapeDtypeStruct((nb, 8, 2 * C), jnp.float32),
        ],
        compiler_params=par,
    )(xe, wc)

    z2, s2b = pl.pallas_call(
        functools.partial(_conv2_kernel, Nb=Nb, Ho=Ho, Wo=Wo, C=C, inv_m=inv_m),
        grid=(nb,),
        in_specs=[
            pl.BlockSpec((R, C), lambda i: (i, 0)),
            pl.BlockSpec((9, C, C), lambda i: (0, 0, 0)),
            pl.BlockSpec((8, C), lambda i: (0, 0)),
            pl.BlockSpec((nb, 8, C), lambda i: (0, 0, 0)),
        ],
        out_specs=[
            pl.BlockSpec((R, C), lambda i: (i, 0)),
            pl.BlockSpec((1, 8, C), lambda i: (i, 0, 0)),
        ],
        out_shape=[
            jax.ShapeDtypeStruct((M, C), jnp.bfloat16),
            jax.ShapeDtypeStruct((nb, 8, C), jnp.float32),
        ],
        scratch_shapes=[pltpu.VMEM((Nb, Ho + 2, Wo + 2, C), jnp.bfloat16)],
        compiler_params=par,
    )(z, w2p, bn, s1b)

    out_flat = pl.pallas_call(
        functools.partial(_final_kernel, inv_m=inv_m),
        grid=(nb,),
        in_specs=[
            pl.BlockSpec((R, C), lambda i: (i, 0)),
            pl.BlockSpec((R, C), lambda i: (i, 1)),
            pl.BlockSpec((nb, 8, C), lambda i: (0, 0, 1)),
            pl.BlockSpec((nb, 8, C), lambda i: (0, 0, 0)),
            pl.BlockSpec((8, C), lambda i: (0, 0)),
        ],
        out_specs=pl.BlockSpec((R, C), lambda i: (i, 0)),
        out_shape=jax.ShapeDtypeStruct((M, C), jnp.float32),
        compiler_params=par,
    )(z2, z, s1b, s2b, bn)

    return out_flat.reshape(N, Ho, Wo, C).transpose(0, 3, 1, 2)


def kernel(x, w1, g1, b1, w2, g2, b2, wd, gd, bd):
    return _block_impl(x, w1, g1, b1, w2, g2, b2, wd, gd, bd)


# speedup vs baseline: 9.5643x; 8.8138x over previous
import functools

import jax
import jax.numpy as jnp
from jax.experimental import pallas as pl
from jax.experimental.pallas import tpu as pltpu

EPS = 1e-5
_VMEM_LIMIT = 100 * 1024 * 1024


def _conv1_kernel(xt_ref, wc_ref, z_ref, st_ref, xpad_ref, *, Nb, Ho, Wo, Cin):
    H, W = 2 * Ho, 2 * Wo
    Ap = Ho + 1
    R = Nb * Ho * Wo
    xpad_ref[...] = jnp.zeros_like(xpad_ref)
    xpad_ref[:, 1:1 + H, 1:1 + W, :] = xt_ref[...]
    x5 = xpad_ref[...].reshape(Nb, Ap, 2, Ap, 2, Cin)
    z = None
    for dy in range(3):
        for dx in range(3):
            tap = dy * 3 + dx
            r0 = dy // 2
            c0 = dx // 2
            t = x5[:, r0:r0 + Ho, dy % 2, c0:c0 + Wo, dx % 2, :].reshape(R, Cin)
            d = jnp.dot(t, wc_ref[tap], preferred_element_type=jnp.float32)
            z = d if z is None else z + d
    z_ref[...] = z.astype(z_ref.dtype)
    st_ref[0, 0:1, :] = jnp.sum(z, axis=0, keepdims=True)
    st_ref[0, 1:2, :] = jnp.sum(z * z, axis=0, keepdims=True)
    st_ref[0, 2:8, :] = jnp.zeros_like(st_ref[0, 2:8, :])


def _conv2_kernel(z_ref, w2_ref, bn_ref, s1_ref, z2_ref, st_ref, ypad_ref,
                  *, Nb, Ho, Wo, C, inv_m):
    R = Nb * Ho * Wo
    s1 = jnp.sum(s1_ref[...], axis=0)
    mean = s1[0:1, :] * inv_m
    var = jnp.maximum(s1[1:2, :] * inv_m - mean * mean, 0.0)
    scale = jax.lax.rsqrt(var + EPS) * bn_ref[0:1, :]
    shift = bn_ref[1:2, :] - mean * scale
    y1 = jnp.maximum(z_ref[...].astype(jnp.float32) * scale + shift, 0.0)

    ypad_ref[...] = jnp.zeros_like(ypad_ref)
    ypad_ref[:, 1:1 + Ho, 1:1 + Wo, :] = (
        y1.astype(ypad_ref.dtype).reshape(Nb, Ho, Wo, C))

    z2 = None
    for dy in range(3):
        for dx in range(3):
            t = ypad_ref[:, dy:dy + Ho, dx:dx + Wo, :].reshape(R, C)
            d = jnp.dot(t, w2_ref[dy * 3 + dx],
                        preferred_element_type=jnp.float32)
            z2 = d if z2 is None else z2 + d
    z2_ref[...] = z2.astype(z2_ref.dtype)
    st_ref[0, 0:1, :] = jnp.sum(z2, axis=0, keepdims=True)
    st_ref[0, 1:2, :] = jnp.sum(z2 * z2, axis=0, keepdims=True)
    st_ref[0, 2:8, :] = jnp.zeros_like(st_ref[0, 2:8, :])


def _final_kernel(z2_ref, zd_ref, s1_ref, s2_ref, bn_ref, o_ref, *, inv_m):
    s1 = jnp.sum(s1_ref[...], axis=0)
    s2 = jnp.sum(s2_ref[...], axis=0)

    m2 = s2[0:1, :] * inv_m
    v2 = jnp.maximum(s2[1:2, :] * inv_m - m2 * m2, 0.0)
    sc2 = jax.lax.rsqrt(v2 + EPS) * bn_ref[2:3, :]
    y2 = z2_ref[...].astype(jnp.float32) * sc2 + (bn_ref[3:4, :] - m2 * sc2)

    md = s1[0:1, :] * inv_m
    vd = jnp.maximum(s1[1:2, :] * inv_m - md * md, 0.0)
    scd = jax.lax.rsqrt(vd + EPS) * bn_ref[4:5, :]
    yd = zd_ref[...].astype(jnp.float32) * scd + (bn_ref[5:6, :] - md * scd)

    o_ref[...] = jnp.maximum(y2 + yd, 0.0)


def _block_impl(x, w1, g1, b1, w2, g2, b2, wd, gd, bd):
    N, Cin, H, W = x.shape
    C = w1.shape[-1]
    assert H % 2 == 0 and W % 2 == 0
    assert Cin % 128 == 0 and C % 128 == 0
    Ho, Wo = H // 2, W // 2
    Ap = Ho + 1
    M = N * Ho * Wo
    inv_m = 1.0 / M
    Nb = next(t for t in (8, 4, 2, 1) if N % t == 0)
    nb = N // Nb
    R = Nb * Ho * Wo

    xt = jnp.transpose(x.astype(jnp.bfloat16), (0, 2, 3, 1))

    wc = jnp.zeros((9, Cin, 2 * C), jnp.float32)
    wc = wc.at[:, :, :C].set(w1.reshape(9, Cin, C))
    wc = wc.at[4, :, C:].set(wd.reshape(Cin, C))
    wc = wc.astype(jnp.bfloat16)
    w2p = w2.reshape(9, C, C).astype(jnp.bfloat16)
    zrow = jnp.zeros_like(g1)
    bn = jnp.stack([g1, b1, g2, b2, gd, bd, zrow, zrow])

    par = pltpu.CompilerParams(dimension_semantics=("parallel",),
                               vmem_limit_bytes=_VMEM_LIMIT)

    z, s1b = pl.pallas_call(
        functools.partial(_conv1_kernel, Nb=Nb, Ho=Ho, Wo=Wo, Cin=Cin),
        grid=(nb,),
        in_specs=[
            pl.BlockSpec((Nb, H, W, Cin), lambda i: (i, 0, 0, 0)),
            pl.BlockSpec((9, Cin, 2 * C), lambda i: (0, 0, 0)),
        ],
        out_specs=[
            pl.BlockSpec((R, 2 * C), lambda i: (i, 0)),
            pl.BlockSpec((1, 8, 2 * C), lambda i: (i, 0, 0)),
        ],
        out_shape=[
            jax.ShapeDtypeStruct((M, 2 * C), jnp.bfloat16),
            jax.ShapeDtypeStruct((nb, 8, 2 * C), jnp.float32),
        ],
        scratch_shapes=[pltpu.VMEM((Nb, 2 * Ap, 2 * Ap, Cin), jnp.bfloat16)],
        compiler_params=par,
    )(xt, wc)

    z2, s2b = pl.pallas_call(
        functools.partial(_conv2_kernel, Nb=Nb, Ho=Ho, Wo=Wo, C=C, inv_m=inv_m),
        grid=(nb,),
        in_specs=[
            pl.BlockSpec((R, C), lambda i: (i, 0)),
            pl.BlockSpec((9, C, C), lambda i: (0, 0, 0)),
            pl.BlockSpec((8, C), lambda i: (0, 0)),
            pl.BlockSpec((nb, 8, C), lambda i: (0, 0, 0)),
        ],
        out_specs=[
            pl.BlockSpec((R, C), lambda i: (i, 0)),
            pl.BlockSpec((1, 8, C), lambda i: (i, 0, 0)),
        ],
        out_shape=[
            jax.ShapeDtypeStruct((M, C), jnp.bfloat16),
            jax.ShapeDtypeStruct((nb, 8, C), jnp.float32),
        ],
        scratch_shapes=[pltpu.VMEM((Nb, Ho + 2, Wo + 2, C), jnp.bfloat16)],
        compiler_params=par,
    )(z, w2p, bn, s1b)

    out_flat = pl.pallas_call(
        functools.partial(_final_kernel, inv_m=inv_m),
        grid=(nb,),
        in_specs=[
            pl.BlockSpec((R, C), lambda i: (i, 0)),
            pl.BlockSpec((R, C), lambda i: (i, 1)),
            pl.BlockSpec((nb, 8, C), lambda i: (0, 0, 1)),
            pl.BlockSpec((nb, 8, C), lambda i: (0, 0, 0)),
            pl.BlockSpec((8, C), lambda i: (0, 0)),
        ],
        out_specs=pl.BlockSpec((R, C), lambda i: (i, 0)),
        out_shape=jax.ShapeDtypeStruct((M, C), jnp.float32),
        compiler_params=par,
    )(z2, z, s1b, s2b, bn)

    return out_flat.reshape(N, Ho, Wo, C).transpose(0, 3, 1, 2)


def kernel(x, w1, g1, b1, w2, g2, b2, wd, gd, bd):
    return _block_impl(x, w1, g1, b1, w2, g2, b2, wd, gd, bd)
